# baseline (speedup 1.0000x reference)
import functools

import jax
import jax.numpy as jnp
from jax import lax
from jax.experimental import pallas as pl
from jax.experimental.pallas import tpu as pltpu
from jax.experimental.pallas import tpu_sc as plsc

_B = 16384
_D = 64
_NC = 2
_NS = 16
_NW = _NC * _NS
_BPW = _B // _NW
_CH = 64
_NCHK = _BPW // _CH
_LANES = 16
_PK = 2


def _transe_body(ent_hbm, rel_hbm, h_hbm, r_hbm, t_hbm, out_hbm,
                 hv, rv, tv, ph, pr, pt, gh, gr, gt, ob, sem):
    wid = lax.axis_index("s") * _NC + lax.axis_index("c")
    base = wid * _BPW

    pltpu.sync_copy(h_hbm.at[pl.ds(base, _BPW)], hv)
    pltpu.sync_copy(r_hbm.at[pl.ds(base, _BPW)], rv)
    pltpu.sync_copy(t_hbm.at[pl.ds(base, _BPW)], tv)

    grp_per_row = _CH // _LANES
    for k in range(_BPW // _LANES):
        s = pl.ds(k * _LANES, _LANES)
        d0 = k // grp_per_row
        d1 = pl.ds((k % grp_per_row) * _LANES, _LANES)
        ph[d0, d1] = lax.shift_right_logical(hv[s], 1)
        pr[d0, d1] = lax.shift_right_logical(rv[s], 1)
        pt[d0, d1] = lax.shift_right_logical(tv[s], 1)

    lanes = lax.iota(jnp.int32, _LANES)
    zero = jnp.zeros((_LANES,), jnp.int32)
    masks = [lanes == l for l in range(_LANES)]

    def chunk_step(j, carry):
        co = j * _CH
        cp_h = pltpu.async_copy(ent_hbm.at[ph.at[j]], gh, sem)
        cp_r = pltpu.async_copy(rel_hbm.at[pr.at[j]], gr, sem)
        cp_t = pltpu.async_copy(ent_hbm.at[pt.at[j]], gt, sem)
        cp_h.wait()
        cp_r.wait()
        cp_t.wait()
        for g in range(_CH // _LANES):
            sl = pl.ds(co + g * _LANES, _LANES)
            qh = hv[sl]
            qr = rv[sl]
            qt = tv[sl]
            for l in range(_LANES):
                oh = (jnp.max(jnp.where(masks[l], qh, zero)) & 1) * _D
                or_ = (jnp.max(jnp.where(masks[l], qr, zero)) & 1) * _D
                ot = (jnp.max(jnp.where(masks[l], qt, zero)) & 1) * _D
                e = g * _LANES + l
                for c in range(_D // _LANES):
                    wh = gh[e, pl.ds(oh + c * _LANES, _LANES)]
                    wr = gr[e, pl.ds(or_ + c * _LANES, _LANES)]
                    wt = gt[e, pl.ds(ot + c * _LANES, _LANES)]
                    ob[e, pl.ds(c * _LANES, _LANES)] = wh + wr - wt
        pltpu.sync_copy(ob, out_hbm.at[pl.ds(base + co, _CH)])
        return carry
    lax.fori_loop(0, _NCHK, chunk_step, 0)


@functools.partial(
    pl.kernel,
    out_type=jax.ShapeDtypeStruct((_B, _D), jnp.float32),
    mesh=plsc.VectorSubcoreMesh(core_axis_name="c", subcore_axis_name="s"),
    compiler_params=pltpu.CompilerParams(needs_layout_passes=False),
    scratch_types=[
        pltpu.VMEM((_BPW,), jnp.int32),
        pltpu.VMEM((_BPW,), jnp.int32),
        pltpu.VMEM((_BPW,), jnp.int32),
        pltpu.VMEM((_NCHK, _CH), jnp.int32),
        pltpu.VMEM((_NCHK, _CH), jnp.int32),
        pltpu.VMEM((_NCHK, _CH), jnp.int32),
        pltpu.VMEM((_CH, _PK * _D), jnp.float32),
        pltpu.VMEM((_CH, _PK * _D), jnp.float32),
        pltpu.VMEM((_CH, _PK * _D), jnp.float32),
        pltpu.VMEM((_CH, _D), jnp.float32),
        pltpu.SemaphoreType.DMA,
    ],
)
def _transe(ent_hbm, rel_hbm, h_hbm, r_hbm, t_hbm, out_hbm,
            hv, rv, tv, ph, pr, pt, gh, gr, gt, ob, sem):
    _transe_body(ent_hbm, rel_hbm, h_hbm, r_hbm, t_hbm, out_hbm,
                 hv, rv, tv, ph, pr, pt, gh, gr, gt, ob, sem)


def kernel(h, r, t, entity_embeddings, relation_embeddings):
    ent_i = lax.optimization_barrier(entity_embeddings)
    rel_i = lax.optimization_barrier(relation_embeddings)
    ent2 = ent_i.reshape(-1, _PK * _D)
    rel2 = rel_i.reshape(-1, _PK * _D)
    return _transe(ent2, rel2, h, r, t)


# speedup vs baseline: 1.6590x; 1.6590x over previous
import functools

import jax
import jax.numpy as jnp
from jax import lax
from jax.experimental import pallas as pl
from jax.experimental.pallas import tpu as pltpu
from jax.experimental.pallas import tpu_sc as plsc

_B = 16384
_D = 64
_NC = 2
_NS = 16
_NW = _NC * _NS
_BPW = _B // _NW
_CH = 32
_NCHK = _BPW // _CH
_LANES = 16


def _transe_body(ent_hbm, rel_hbm, h_hbm, r_hbm, t_hbm, out_hbm,
                 hv, rv, tv, gh, gr, gt, ob, sem):
    wid = lax.axis_index("s") * _NC + lax.axis_index("c")
    base = wid * _BPW

    pltpu.sync_copy(h_hbm.at[pl.ds(base, _BPW)], hv)
    pltpu.sync_copy(r_hbm.at[pl.ds(base, _BPW)], rv)
    pltpu.sync_copy(t_hbm.at[pl.ds(base, _BPW)], tv)

    lanes = lax.iota(jnp.int32, _LANES)
    zero = jnp.zeros((_LANES,), jnp.int32)
    masks = [lanes == l for l in range(_LANES)]

    def chunk_step(j, carry):
        co = j * _CH
        cps = []
        for g in range(_CH // _LANES):
            s = pl.ds(co + g * _LANES, _LANES)
            hvv = hv[s]
            rvv = rv[s]
            tvv = tv[s]
            for l in range(_LANES):
                he = jnp.max(jnp.where(masks[l], hvv, zero))
                re_ = jnp.max(jnp.where(masks[l], rvv, zero))
                te = jnp.max(jnp.where(masks[l], tvv, zero))
                e = g * _LANES + l
                cps.append(pltpu.async_copy(ent_hbm.at[pl.ds(he, 1)],
                                            gh.at[pl.ds(e, 1)], sem))
                cps.append(pltpu.async_copy(rel_hbm.at[pl.ds(re_, 1)],
                                            gr.at[pl.ds(e, 1)], sem))
                cps.append(pltpu.async_copy(ent_hbm.at[pl.ds(te, 1)],
                                            gt.at[pl.ds(e, 1)], sem))
        for cp in cps:
            cp.wait()
        for e in range(_CH):
            for c in range(_D // _LANES):
                cs = pl.ds(c * _LANES, _LANES)
                ob[e, cs] = gh[e, cs] + gr[e, cs] - gt[e, cs]
        pltpu.sync_copy(ob, out_hbm.at[pl.ds(base + co, _CH)])
        return carry
    lax.fori_loop(0, _NCHK, chunk_step, 0)


@functools.partial(
    pl.kernel,
    out_type=jax.ShapeDtypeStruct((_B, _D), jnp.float32),
    mesh=plsc.VectorSubcoreMesh(core_axis_name="c", subcore_axis_name="s"),
    compiler_params=pltpu.CompilerParams(needs_layout_passes=False),
    scratch_types=[
        pltpu.VMEM((_BPW,), jnp.int32),
        pltpu.VMEM((_BPW,), jnp.int32),
        pltpu.VMEM((_BPW,), jnp.int32),
        pltpu.VMEM((_CH, _D), jnp.float32),
        pltpu.VMEM((_CH, _D), jnp.float32),
        pltpu.VMEM((_CH, _D), jnp.float32),
        pltpu.VMEM((_CH, _D), jnp.float32),
        pltpu.SemaphoreType.DMA,
    ],
)
def _transe(ent_hbm, rel_hbm, h_hbm, r_hbm, t_hbm, out_hbm,
            hv, rv, tv, gh, gr, gt, ob, sem):
    _transe_body(ent_hbm, rel_hbm, h_hbm, r_hbm, t_hbm, out_hbm,
                 hv, rv, tv, gh, gr, gt, ob, sem)


def kernel(h, r, t, entity_embeddings, relation_embeddings):
    return _transe(entity_embeddings, relation_embeddings, h, r, t)
